# Initial kernel scaffold; baseline (speedup 1.0000x reference)
import jax
import jax.numpy as jnp
from jax.experimental import pallas as pl
from jax.experimental.pallas import tpu as pltpu

_NEG = jnp.float32(-1e30)


def _flash_body(B, G, D, NB):
    def body(x_ref, seg_ref, wg_ref, out_ref, m_ref, d_ref, acc_ref):
        i = pl.program_id(0)

        @pl.when(i == 0)
        def _init():
            m_ref[...] = jnp.full((1, G), _NEG, jnp.float32)
            d_ref[...] = jnp.zeros((1, G), jnp.float32)
            acc_ref[...] = jnp.zeros((G, D), jnp.float32)

        x = x_ref[...]
        s = seg_ref[...]
        w = wg_ref[...]

        gate = jax.lax.dot_general(x, w, (((1,), (0,)), ((), ())),
                                   preferred_element_type=jnp.float32)

        cols = jax.lax.broadcasted_iota(jnp.int32, (B, G), 1)
        mask = s == cols
        maskf = mask.astype(jnp.float32)

        bm = jnp.max(jnp.where(mask, gate, _NEG), axis=0, keepdims=True)
        m_old = m_ref[...]
        m_new = jnp.maximum(m_old, bm)
        scale = jnp.exp(m_old - m_new)

        m_sel = jax.lax.dot_general(maskf, m_new, (((1,), (1,)), ((), ())),
                                    preferred_element_type=jnp.float32)
        e = jnp.exp(gate - m_sel)

        es = jnp.sum(maskf * e, axis=0, keepdims=True)
        m_ref[...] = m_new
        d_ref[...] = d_ref[...] * scale + es

        rows_g = jax.lax.broadcasted_iota(jnp.int32, (G, G), 0)
        cols_g = jax.lax.broadcasted_iota(jnp.int32, (G, G), 1)
        eye = rows_g == cols_g
        diag_scale = jnp.where(eye, scale, jnp.float32(0.0))

        xe = x * e
        contrib = jax.lax.dot_general(maskf, xe, (((0,), (0,)), ((), ())),
                                      preferred_element_type=jnp.float32)
        acc_ref[...] = jax.lax.dot_general(
            diag_scale, acc_ref[...], (((1,), (0,)), ((), ())),
            preferred_element_type=jnp.float32) + contrib

        @pl.when(i == NB - 1)
        def _fin():
            d = d_ref[...]
            recip = jnp.where(d > 0, jnp.float32(1.0) / d, jnp.float32(0.0))
            diag_r = jnp.where(eye, recip, jnp.float32(0.0))
            out_ref[...] = jax.lax.dot_general(
                diag_r, acc_ref[...], (((1,), (0,)), ((), ())),
                preferred_element_type=jnp.float32)

    return body


def _flash_call(N, D, G, B, interpret=False):
    NB = N // B
    return pl.pallas_call(
        _flash_body(B, G, D, NB),
        grid=(NB,),
        in_specs=[
            pl.BlockSpec((B, D), lambda i: (i, 0)),
            pl.BlockSpec((B, 1), lambda i: (i, 0)),
            pl.BlockSpec((D, 1), lambda i: (0, 0)),
        ],
        out_specs=pl.BlockSpec((G, D), lambda i: (0, 0)),
        out_shape=jax.ShapeDtypeStruct((G, D), jnp.float32),
        scratch_shapes=[
            pltpu.VMEM((1, G), jnp.float32),
            pltpu.VMEM((1, G), jnp.float32),
            pltpu.VMEM((G, D), jnp.float32),
        ],
        interpret=interpret,
    )


def kernel(features, segment_ids, Wg, bg):
    N, D = features.shape
    G = 256
    B = 2000 if N % 2000 == 0 else 8
    seg = segment_ids.astype(jnp.int32).reshape(N, 1)
    return _flash_call(N, D, G, B)(features, seg, Wg)


# speedup vs baseline: 11.7286x; 11.7286x over previous
import jax
import jax.numpy as jnp
from jax.experimental import pallas as pl
from jax.experimental.pallas import tpu as pltpu

_NEG = -1e30


def _flash_body(B, G, D, NB):
    def body(x_ref, seg_ref, wg_ref, out_ref, m_ref, d_ref, acc_ref):
        i = pl.program_id(0)

        @pl.when(i == 0)
        def _init():
            m_ref[...] = jnp.full((1, G), _NEG, jnp.float32)
            d_ref[...] = jnp.zeros((1, G), jnp.float32)
            acc_ref[...] = jnp.zeros((G, D), jnp.float32)

        x = x_ref[...]
        s = seg_ref[...]
        w = wg_ref[...]

        gate = jax.lax.dot_general(x, w, (((1,), (0,)), ((), ())),
                                   preferred_element_type=jnp.float32)

        cols = jax.lax.broadcasted_iota(jnp.int32, (B, G), 1)
        mask = s == cols
        maskf = mask.astype(jnp.float32)

        bm = jnp.max(jnp.where(mask, gate, _NEG), axis=0, keepdims=True)
        m_old = m_ref[...]
        m_new = jnp.maximum(m_old, bm)
        scale = jnp.exp(m_old - m_new)

        m_sel = jax.lax.dot_general(maskf, m_new, (((1,), (1,)), ((), ())),
                                    preferred_element_type=jnp.float32)
        e = jnp.exp(gate - m_sel)

        es = jnp.sum(maskf * e, axis=0, keepdims=True)
        m_ref[...] = m_new
        d_ref[...] = d_ref[...] * scale + es

        rows_g = jax.lax.broadcasted_iota(jnp.int32, (G, G), 0)
        cols_g = jax.lax.broadcasted_iota(jnp.int32, (G, G), 1)
        eye = rows_g == cols_g
        diag_scale = jnp.where(eye, scale, jnp.float32(0.0))

        xe = x * e
        contrib = jax.lax.dot_general(maskf, xe, (((0,), (0,)), ((), ())),
                                      preferred_element_type=jnp.float32)
        acc_ref[...] = jax.lax.dot_general(
            diag_scale, acc_ref[...], (((1,), (0,)), ((), ())),
            preferred_element_type=jnp.float32) + contrib

        @pl.when(i == NB - 1)
        def _fin():
            d = d_ref[...]
            recip = jnp.where(d > 0, jnp.float32(1.0) / d, jnp.float32(0.0))
            diag_r = jnp.where(eye, recip, jnp.float32(0.0))
            out_ref[...] = jax.lax.dot_general(
                diag_r, acc_ref[...], (((1,), (0,)), ((), ())),
                preferred_element_type=jnp.float32)

    return body


def _flash_call(N, D, G, B, interpret=False):
    NB = N // B
    return pl.pallas_call(
        _flash_body(B, G, D, NB),
        grid=(NB,),
        in_specs=[
            pl.BlockSpec((B, D), lambda i: (i, 0)),
            pl.BlockSpec((B, 1), lambda i: (i, 0)),
            pl.BlockSpec((D, 1), lambda i: (0, 0)),
        ],
        out_specs=pl.BlockSpec((G, D), lambda i: (0, 0)),
        out_shape=jax.ShapeDtypeStruct((G, D), jnp.float32),
        scratch_shapes=[
            pltpu.VMEM((1, G), jnp.float32),
            pltpu.VMEM((1, G), jnp.float32),
            pltpu.VMEM((G, D), jnp.float32),
        ],
        interpret=interpret,
    )


def kernel(features, segment_ids, Wg, bg):
    N, D = features.shape
    G = 256
    B = 2000 if N % 2000 == 0 else 8
    seg = segment_ids.astype(jnp.int32).reshape(N, 1)
    return _flash_call(N, D, G, B)(features, seg, Wg)


# speedup vs baseline: 20.9822x; 1.7890x over previous
import jax
import jax.numpy as jnp
from jax.experimental import pallas as pl
from jax.experimental.pallas import tpu as pltpu

_NEG = -1e30


def _flash_body(B, G, D, NB):
    def body(x_ref, seg_ref, wg_ref, out_ref, m_ref, d_ref, acc_ref):
        i = pl.program_id(0)

        @pl.when(i == 0)
        def _init():
            m_ref[...] = jnp.full((G, 1), _NEG, jnp.float32)
            d_ref[...] = jnp.zeros((G, 1), jnp.float32)
            acc_ref[...] = jnp.zeros((G, D), jnp.float32)

        x = x_ref[...]
        s = seg_ref[0]
        w = wg_ref[...]

        gateT = jax.lax.dot_general(w, x, (((0,), (1,)), ((), ())),
                                    preferred_element_type=jnp.float32)

        rows = jax.lax.broadcasted_iota(jnp.int32, (G, B), 0)
        maskT = rows == s
        maskf = maskT.astype(jnp.float32)

        gmask = jnp.where(maskT, gateT, _NEG)
        bm = jnp.max(gmask, axis=1, keepdims=True)
        m_old = m_ref[...]
        m_new = jnp.maximum(m_old, bm)
        scale = jnp.exp(m_old - m_new)

        m_selT = jax.lax.dot_general(m_new, maskf, (((0,), (0,)), ((), ())),
                                     preferred_element_type=jnp.float32)
        eT = jnp.exp(gateT - m_selT)

        me = maskf * eT
        ones_b = jnp.ones((B, 1), jnp.float32)
        es = jax.lax.dot_general(me, ones_b, (((1,), (0,)), ((), ())),
                                 preferred_element_type=jnp.float32)
        m_ref[...] = m_new
        d_ref[...] = d_ref[...] * scale + es

        contrib = jax.lax.dot_general(me, x, (((1,), (0,)), ((), ())),
                                      preferred_element_type=jnp.float32)
        acc_ref[...] = acc_ref[...] * scale + contrib

        @pl.when(i == NB - 1)
        def _fin():
            d = d_ref[...]
            recip = jnp.where(d > 0, jnp.float32(1.0) / d, jnp.float32(0.0))
            out_ref[...] = acc_ref[...] * recip

    return body


def _flash_call(N, D, G, B, interpret=False):
    NB = N // B
    return pl.pallas_call(
        _flash_body(B, G, D, NB),
        grid=(NB,),
        in_specs=[
            pl.BlockSpec((B, D), lambda i: (i, 0)),
            pl.BlockSpec((1, 1, B), lambda i: (i, 0, 0)),
            pl.BlockSpec((D, 1), lambda i: (0, 0)),
        ],
        out_specs=pl.BlockSpec((G, D), lambda i: (0, 0)),
        out_shape=jax.ShapeDtypeStruct((G, D), jnp.float32),
        scratch_shapes=[
            pltpu.VMEM((G, 1), jnp.float32),
            pltpu.VMEM((G, 1), jnp.float32),
            pltpu.VMEM((G, D), jnp.float32),
        ],
        interpret=interpret,
    )


def kernel(features, segment_ids, Wg, bg):
    N, D = features.shape
    G = 256
    B = 2000 if N % 2000 == 0 else 8
    seg = segment_ids.astype(jnp.int32).reshape(N // B, 1, B)
    return _flash_call(N, D, G, B)(features, seg, Wg)


# speedup vs baseline: 24.1886x; 1.1528x over previous
import jax
import jax.numpy as jnp
from jax.experimental import pallas as pl
from jax.experimental.pallas import tpu as pltpu

_NEG = -1e30


def _flash_body(B, G, D, NB):
    def body(x_ref, seg_ref, wg_ref, out_ref, m_ref, d_ref, acc_ref):
        i = pl.program_id(0)

        @pl.when(i == 0)
        def _init():
            m_ref[...] = jnp.full((G, 1), _NEG, jnp.float32)
            d_ref[...] = jnp.zeros((G, 1), jnp.float32)
            acc_ref[...] = jnp.zeros((G, D), jnp.float32)

        x = x_ref[...]
        s = seg_ref[0]
        w = wg_ref[...]

        gateT = jax.lax.dot_general(w, x, (((0,), (1,)), ((), ())),
                                    preferred_element_type=jnp.float32)

        rows = jax.lax.broadcasted_iota(jnp.int32, (G, B), 0)
        maskT = rows == s
        maskf = maskT.astype(jnp.float32)

        gmask = jnp.where(maskT, gateT, _NEG)
        bm = jnp.max(gmask, axis=1, keepdims=True)
        m_old = m_ref[...]
        m_new = jnp.maximum(m_old, bm)
        scale = jnp.exp(m_old - m_new)

        m_selT = jax.lax.dot_general(m_new, maskf, (((0,), (0,)), ((), ())),
                                     preferred_element_type=jnp.float32)
        eT = jnp.exp(gateT - m_selT)

        me = maskf * eT
        ones_b = jnp.ones((B, 1), jnp.float32)
        es = jax.lax.dot_general(me, ones_b, (((1,), (0,)), ((), ())),
                                 preferred_element_type=jnp.float32)
        m_ref[...] = m_new
        d_ref[...] = d_ref[...] * scale + es

        contrib = jax.lax.dot_general(me, x, (((1,), (0,)), ((), ())),
                                      preferred_element_type=jnp.float32)
        acc_ref[...] = acc_ref[...] * scale + contrib

        @pl.when(i == NB - 1)
        def _fin():
            d = d_ref[...]
            recip = jnp.where(d > 0, jnp.float32(1.0) / d, jnp.float32(0.0))
            out_ref[...] = acc_ref[...] * recip

    return body


def _flash_call(N, D, G, B, interpret=False):
    NB = N // B
    return pl.pallas_call(
        _flash_body(B, G, D, NB),
        grid=(NB,),
        in_specs=[
            pl.BlockSpec((B, D), lambda i: (i, 0)),
            pl.BlockSpec((1, 1, B), lambda i: (i, 0, 0)),
            pl.BlockSpec((D, 1), lambda i: (0, 0)),
        ],
        out_specs=pl.BlockSpec((G, D), lambda i: (0, 0)),
        out_shape=jax.ShapeDtypeStruct((G, D), jnp.float32),
        scratch_shapes=[
            pltpu.VMEM((G, 1), jnp.float32),
            pltpu.VMEM((G, 1), jnp.float32),
            pltpu.VMEM((G, D), jnp.float32),
        ],
        interpret=interpret,
    )


def kernel(features, segment_ids, Wg, bg):
    N, D = features.shape
    G = 256
    B = 4000 if N % 4000 == 0 else 8
    seg = segment_ids.astype(jnp.int32).reshape(N // B, 1, B)
    return _flash_call(N, D, G, B)(features, seg, Wg)


# speedup vs baseline: 25.0017x; 1.0336x over previous
import jax
import jax.numpy as jnp
from jax.experimental import pallas as pl
from jax.experimental.pallas import tpu as pltpu

_NEG = -1e30


def _flash_body(B, G, D, NB):
    def body(x_ref, seg_ref, wg_ref, out_ref, m_ref, d_ref, acc_ref):
        i = pl.program_id(0)

        @pl.when(i == 0)
        def _init():
            m_ref[...] = jnp.full((G, 1), _NEG, jnp.float32)
            d_ref[...] = jnp.zeros((G, 1), jnp.float32)
            acc_ref[...] = jnp.zeros((G, D), jnp.float32)

        x = x_ref[...]
        s = seg_ref[0]
        w = wg_ref[...]

        gateT = jax.lax.dot_general(w, x, (((0,), (1,)), ((), ())),
                                    preferred_element_type=jnp.float32)

        rows = jax.lax.broadcasted_iota(jnp.int32, (G, B), 0)
        maskT = rows == s
        maskf = maskT.astype(jnp.float32)

        gmask = jnp.where(maskT, gateT, _NEG)
        bm = jnp.max(gmask, axis=1, keepdims=True)
        m_old = m_ref[...]
        m_new = jnp.maximum(m_old, bm)
        scale = jnp.exp(m_old - m_new)

        m_selT = jax.lax.dot_general(m_new, maskf, (((0,), (0,)), ((), ())),
                                     preferred_element_type=jnp.float32)
        eT = jnp.exp(gateT - m_selT)

        me = maskf * eT
        ones_b = jnp.ones((B, 1), jnp.float32)
        es = jax.lax.dot_general(me, ones_b, (((1,), (0,)), ((), ())),
                                 preferred_element_type=jnp.float32)
        m_ref[...] = m_new
        d_ref[...] = d_ref[...] * scale + es

        contrib = jax.lax.dot_general(me, x, (((1,), (0,)), ((), ())),
                                      preferred_element_type=jnp.float32)
        acc_ref[...] = acc_ref[...] * scale + contrib

        @pl.when(i == NB - 1)
        def _fin():
            d = d_ref[...]
            recip = jnp.where(d > 0, jnp.float32(1.0) / d, jnp.float32(0.0))
            out_ref[...] = acc_ref[...] * recip

    return body


def _flash_call(N, D, G, B, interpret=False):
    NB = N // B
    return pl.pallas_call(
        _flash_body(B, G, D, NB),
        grid=(NB,),
        in_specs=[
            pl.BlockSpec((B, D), lambda i: (i, 0)),
            pl.BlockSpec((1, 1, B), lambda i: (i, 0, 0)),
            pl.BlockSpec((D, 1), lambda i: (0, 0)),
        ],
        out_specs=pl.BlockSpec((G, D), lambda i: (0, 0)),
        out_shape=jax.ShapeDtypeStruct((G, D), jnp.float32),
        scratch_shapes=[
            pltpu.VMEM((G, 1), jnp.float32),
            pltpu.VMEM((G, 1), jnp.float32),
            pltpu.VMEM((G, D), jnp.float32),
        ],
        interpret=interpret,
    )


def kernel(features, segment_ids, Wg, bg):
    N, D = features.shape
    G = 256
    B = 5000 if N % 5000 == 0 else 8
    seg = segment_ids.astype(jnp.int32).reshape(N // B, 1, B)
    return _flash_call(N, D, G, B)(features, seg, Wg)


# speedup vs baseline: 26.6407x; 1.0656x over previous
import jax
import jax.numpy as jnp
from jax.experimental import pallas as pl
from jax.experimental.pallas import tpu as pltpu

_NEG = -1e30


def _flash_body(B, G, D, NB):
    def body(x_ref, seg_ref, wg_ref, out_ref, m_ref, d_ref, acc_ref):
        i = pl.program_id(0)

        @pl.when(i == 0)
        def _init():
            m_ref[...] = jnp.full((G, 1), _NEG, jnp.float32)
            d_ref[...] = jnp.zeros((G, 1), jnp.float32)
            acc_ref[...] = jnp.zeros((G, D), jnp.float32)

        x = x_ref[...]
        s = seg_ref[0]
        w = wg_ref[...]

        gateT = jax.lax.dot_general(w, x, (((0,), (1,)), ((), ())),
                                    preferred_element_type=jnp.float32)

        rows = jax.lax.broadcasted_iota(jnp.int32, (G, B), 0)
        maskT = rows == s
        maskf = maskT.astype(jnp.float32)

        gmask = jnp.where(maskT, gateT, _NEG)
        bm = jnp.max(gmask, axis=1, keepdims=True)
        m_old = m_ref[...]
        m_new = jnp.maximum(m_old, bm)
        scale = jnp.exp(m_old - m_new)

        m_selT = jax.lax.dot_general(m_new, maskf, (((0,), (0,)), ((), ())),
                                     preferred_element_type=jnp.float32)
        eT = jnp.exp(gateT - m_selT)

        me = maskf * eT
        ones_b = jnp.ones((B, 1), jnp.float32)
        es = jax.lax.dot_general(me, ones_b, (((1,), (0,)), ((), ())),
                                 preferred_element_type=jnp.float32)
        m_ref[...] = m_new
        d_ref[...] = d_ref[...] * scale + es

        contrib = jax.lax.dot_general(me, x, (((1,), (0,)), ((), ())),
                                      preferred_element_type=jnp.float32)
        acc_ref[...] = acc_ref[...] * scale + contrib

        @pl.when(i == NB - 1)
        def _fin():
            d = d_ref[...]
            recip = jnp.where(d > 0, jnp.float32(1.0) / d, jnp.float32(0.0))
            out_ref[...] = acc_ref[...] * recip

    return body


def _flash_call(N, D, G, B, interpret=False):
    NB = N // B
    return pl.pallas_call(
        _flash_body(B, G, D, NB),
        grid=(NB,),
        in_specs=[
            pl.BlockSpec((B, D), lambda i: (i, 0)),
            pl.BlockSpec((1, 1, B), lambda i: (i, 0, 0)),
            pl.BlockSpec((D, 1), lambda i: (0, 0)),
        ],
        out_specs=pl.BlockSpec((G, D), lambda i: (0, 0)),
        out_shape=jax.ShapeDtypeStruct((G, D), jnp.float32),
        scratch_shapes=[
            pltpu.VMEM((G, 1), jnp.float32),
            pltpu.VMEM((G, 1), jnp.float32),
            pltpu.VMEM((G, D), jnp.float32),
        ],
        interpret=interpret,
    )


def kernel(features, segment_ids, Wg, bg):
    N, D = features.shape
    G = 256
    B = 10000 if N % 10000 == 0 else 8
    seg = segment_ids.astype(jnp.int32).reshape(N // B, 1, B)
    return _flash_call(N, D, G, B)(features, seg, Wg)


# speedup vs baseline: 26.9590x; 1.0120x over previous
import jax
import jax.numpy as jnp
from jax.experimental import pallas as pl
from jax.experimental.pallas import tpu as pltpu

_NEG = -1e30


def _flash_body(B, G, D, NB):
    def body(x_ref, seg_ref, wg_ref, out_ref, m_ref, d_ref, acc_ref):
        i = pl.program_id(0)

        @pl.when(i == 0)
        def _init():
            m_ref[...] = jnp.full((G, 1), _NEG, jnp.float32)
            d_ref[...] = jnp.zeros((G, 1), jnp.float32)
            acc_ref[...] = jnp.zeros((G, D), jnp.float32)

        x = x_ref[...]
        s = seg_ref[0]
        w = wg_ref[...]

        gateT = jax.lax.dot_general(w, x, (((0,), (1,)), ((), ())),
                                    preferred_element_type=jnp.float32)

        rows = jax.lax.broadcasted_iota(jnp.int32, (G, B), 0)
        maskT = rows == s
        maskf = maskT.astype(jnp.float32)

        gmask = jnp.where(maskT, gateT, _NEG)
        bm = jnp.max(gmask, axis=1, keepdims=True)
        m_old = m_ref[...]
        m_new = jnp.maximum(m_old, bm)
        scale = jnp.exp(m_old - m_new)

        m_selT = jax.lax.dot_general(m_new, maskf, (((0,), (0,)), ((), ())),
                                     preferred_element_type=jnp.float32)
        eT = jnp.exp(gateT - m_selT)

        me = maskf * eT
        ones_b = jnp.ones((B, 1), jnp.float32)
        es = jax.lax.dot_general(me, ones_b, (((1,), (0,)), ((), ())),
                                 preferred_element_type=jnp.float32)
        m_ref[...] = m_new
        d_ref[...] = d_ref[...] * scale + es

        contrib = jax.lax.dot_general(me, x, (((1,), (0,)), ((), ())),
                                      preferred_element_type=jnp.float32)
        acc_ref[...] = acc_ref[...] * scale + contrib

        @pl.when(i == NB - 1)
        def _fin():
            d = d_ref[...]
            recip = jnp.where(d > 0, jnp.float32(1.0) / d, jnp.float32(0.0))
            out_ref[...] = acc_ref[...] * recip

    return body


def _flash_call(N, D, G, B, interpret=False):
    NB = N // B
    return pl.pallas_call(
        _flash_body(B, G, D, NB),
        grid=(NB,),
        in_specs=[
            pl.BlockSpec((B, D), lambda i: (i, 0)),
            pl.BlockSpec((1, 1, B), lambda i: (i, 0, 0)),
            pl.BlockSpec((D, 1), lambda i: (0, 0)),
        ],
        out_specs=pl.BlockSpec((G, D), lambda i: (0, 0)),
        out_shape=jax.ShapeDtypeStruct((G, D), jnp.float32),
        scratch_shapes=[
            pltpu.VMEM((G, 1), jnp.float32),
            pltpu.VMEM((G, 1), jnp.float32),
            pltpu.VMEM((G, D), jnp.float32),
        ],
        interpret=interpret,
    )


def kernel(features, segment_ids, Wg, bg):
    N, D = features.shape
    G = 256
    B = 20000 if N % 20000 == 0 else 8
    seg = segment_ids.astype(jnp.int32).reshape(N // B, 1, B)
    return _flash_call(N, D, G, B)(features, seg, Wg)
